# Initial kernel scaffold; baseline (speedup 1.0000x reference)
import jax
import jax.numpy as jnp
from jax.experimental import pallas as pl


def kernel(h, edge_index, edge_attr, W, Wf, bf, a):
    raise NotImplementedError("write your pallas kernel here")


# speedup vs baseline: 12.5125x; 12.5125x over previous
import jax
import jax.numpy as jnp
from jax import lax
from jax.experimental import pallas as pl
from jax.experimental.pallas import tpu as pltpu
from jax.experimental.pallas import tpu_sc as plsc

N = 10000
E = 320000
IN_DIM = 128
OUT_DIM = 32
NUM_HEADS = 4
FEAT = 4

N2 = 10240
NW = 32
ESUB = 10240
EPAD = NW * ESUB
C = 512
K = C // 128
NB = 1024
EB = EPAD // 10


def _prep_node_body(h_ref, w_ref, a_ref, z_ref, es_ref, ed_ref):
    hb = h_ref[...]
    for i in range(NUM_HEADS):
        z = jnp.dot(hb, w_ref[i], preferred_element_type=jnp.float32)
        z_ref[i] = z
        a1 = a_ref[i, 0:OUT_DIM]
        a2 = a_ref[i, OUT_DIM:2 * OUT_DIM]
        es_ref[i] = jnp.sum(z * a1[None, :], axis=1)
        ed_ref[i] = jnp.sum(z * a2[None, :], axis=1)


def _prep_node(h_p, W, a):
    return pl.pallas_call(
        _prep_node_body,
        grid=(N2 // NB,),
        in_specs=[
            pl.BlockSpec((NB, IN_DIM), lambda r: (r, 0)),
            pl.BlockSpec((NUM_HEADS, IN_DIM, OUT_DIM), lambda r: (0, 0, 0)),
            pl.BlockSpec((NUM_HEADS, 3 * OUT_DIM), lambda r: (0, 0)),
        ],
        out_specs=(
            pl.BlockSpec((NUM_HEADS, NB, OUT_DIM), lambda r: (0, r, 0)),
            pl.BlockSpec((NUM_HEADS, NB), lambda r: (0, r)),
            pl.BlockSpec((NUM_HEADS, NB), lambda r: (0, r)),
        ),
        out_shape=(
            jax.ShapeDtypeStruct((NUM_HEADS, N2, OUT_DIM), jnp.float32),
            jax.ShapeDtypeStruct((NUM_HEADS, N2), jnp.float32),
            jax.ShapeDtypeStruct((NUM_HEADS, N2), jnp.float32),
        ),
    )(h_p, W, a)


def _prep_edge_body(eat_ref, wf_ref, bf_ref, a_ref, ef_ref):
    for i in range(NUM_HEADS):
        a3 = a_ref[i, 2 * OUT_DIM:3 * OUT_DIM]
        ef = jnp.full((EB,), jnp.sum(bf_ref[i] * a3), jnp.float32)
        for k in range(FEAT):
            ef = ef + eat_ref[k] * jnp.sum(wf_ref[i, k] * a3)
        ef_ref[i] = ef


def _prep_edge(ea_t, Wf, bf, a):
    return pl.pallas_call(
        _prep_edge_body,
        grid=(EPAD // EB,),
        in_specs=[
            pl.BlockSpec((FEAT, EB), lambda b: (0, b)),
            pl.BlockSpec((NUM_HEADS, FEAT, OUT_DIM), lambda b: (0, 0, 0)),
            pl.BlockSpec((NUM_HEADS, OUT_DIM), lambda b: (0, 0)),
            pl.BlockSpec((NUM_HEADS, 3 * OUT_DIM), lambda b: (0, 0)),
        ],
        out_specs=pl.BlockSpec((NUM_HEADS, EB), lambda b: (0, b)),
        out_shape=jax.ShapeDtypeStruct((NUM_HEADS, EPAD), jnp.float32),
    )(ea_t, Wf, bf, a)


def _sc_body(src_hbm, dst_hbm, ef_hbm, es_hbm, ed_hbm, z_hbm,
             zacc_hbm, zs_hbm, acc_out, s_out,
             es_v, ed_v, srcb, dstb, efb, exb, rows, gidx2, didx2,
             acc_sh, s_sh, sem):
    cid = lax.axis_index("c")
    sid = lax.axis_index("s")
    wid = sid * 2 + cid

    @pl.when(sid == 0)
    def _():
        pltpu.sync_copy(zacc_hbm, acc_sh)
        pltpu.sync_copy(zs_hbm, s_sh)
    plsc.subcore_barrier()

    def head_body(hh, carry):
        pltpu.sync_copy(es_hbm.at[hh], es_v)
        pltpu.sync_copy(ed_hbm.at[hh], ed_v)

        def chunk_body(ch, carry2):
            base = wid * ESUB + ch * C
            pltpu.sync_copy(src_hbm.at[pl.ds(base, C)], srcb)
            pltpu.sync_copy(dst_hbm.at[pl.ds(base, C)], dstb)
            pltpu.sync_copy(ef_hbm.at[pl.ds(hh * EPAD + base, C)], efb)

            def j_body(j, carry3):
                ex_regs = []
                for g in range(8):
                    off = j * 128 + g * 16
                    s16 = srcb[pl.ds(off, 16)]
                    d16 = dstb[pl.ds(off, 16)]
                    e = (plsc.load_gather(es_v, [s16])
                         + plsc.load_gather(ed_v, [d16])
                         + efb[pl.ds(off, 16)])
                    e = jnp.where(e < 0.0, e * 0.2, e)
                    ex = jnp.exp(e)
                    ex_regs.append(ex)
                    exb[pl.ds(off, 16)] = ex
                    jlane = jnp.full((16,), j, jnp.int32)
                    lane = lax.iota(jnp.int32, 16) + (g * 16)
                    plsc.store_scatter(gidx2, [jlane, lane], s16 + hh * N2)
                    plsc.store_scatter(didx2, [jlane, lane], d16 + hh * N2)
                pltpu.async_copy(z_hbm.at[gidx2.at[j]],
                                 rows.at[pl.ds(j * 128, 128)], sem).wait()
                for g in range(8):
                    row16 = lax.iota(jnp.int32, 16) + (j * 128 + g * 16)
                    exg = ex_regs[g]
                    for cc in range(OUT_DIM):
                        c16 = jnp.full((16,), cc, jnp.int32)
                        v = plsc.load_gather(rows, [row16, c16])
                        plsc.store_scatter(rows, [row16, c16], v * exg)
                pltpu.sync_copy(rows.at[pl.ds(j * 128, 128)],
                                acc_sh.at[didx2.at[j]], add=True)
                pltpu.sync_copy(exb.at[pl.ds(j * 128, 128)],
                                s_sh.at[didx2.at[j]], add=True)
                return carry3

            return lax.fori_loop(0, K, j_body, carry2)

        return lax.fori_loop(0, ESUB // C, chunk_body, carry)

    lax.fori_loop(0, NUM_HEADS, head_body, 0)
    plsc.subcore_barrier()

    @pl.when(sid == 0)
    def _():
        pltpu.sync_copy(acc_sh, acc_out.at[cid])
        pltpu.sync_copy(s_sh, s_out.at[cid])


_sc_edge = pl.kernel(
    _sc_body,
    out_type=(
        jax.ShapeDtypeStruct((2, NUM_HEADS * N2, OUT_DIM), jnp.float32),
        jax.ShapeDtypeStruct((2, NUM_HEADS * N2), jnp.float32),
    ),
    mesh=plsc.VectorSubcoreMesh(core_axis_name="c", subcore_axis_name="s"),
    compiler_params=pltpu.CompilerParams(needs_layout_passes=False,
                                         use_tc_tiling_on_sc=False),
    scratch_types=[
        pltpu.VMEM((N2,), jnp.float32),
        pltpu.VMEM((N2,), jnp.float32),
        pltpu.VMEM((C,), jnp.int32),
        pltpu.VMEM((C,), jnp.int32),
        pltpu.VMEM((C,), jnp.float32),
        pltpu.VMEM((C,), jnp.float32),
        pltpu.VMEM((C, OUT_DIM), jnp.float32),
        pltpu.VMEM((K, 128), jnp.int32),
        pltpu.VMEM((K, 128), jnp.int32),
        pltpu.VMEM_SHARED((NUM_HEADS * N2, OUT_DIM), jnp.float32),
        pltpu.VMEM_SHARED((NUM_HEADS * N2,), jnp.float32),
        pltpu.SemaphoreType.DMA,
    ],
)


def _finish_body(acc_ref, s_ref, out_ref):
    for i in range(NUM_HEADS):
        num = acc_ref[0, i] + acc_ref[1, i]
        den = s_ref[0, i] + s_ref[1, i]
        out_ref[:, i * OUT_DIM:(i + 1) * OUT_DIM] = (
            num / (den + 1e-9)[:, None])


def _finish(acc, s):
    return pl.pallas_call(
        _finish_body,
        grid=(N2 // NB,),
        in_specs=[
            pl.BlockSpec((2, NUM_HEADS, NB, OUT_DIM), lambda r: (0, 0, r, 0)),
            pl.BlockSpec((2, NUM_HEADS, NB), lambda r: (0, 0, r)),
        ],
        out_specs=pl.BlockSpec((NB, NUM_HEADS * OUT_DIM), lambda r: (r, 0)),
        out_shape=jax.ShapeDtypeStruct((N2, NUM_HEADS * OUT_DIM),
                                       jnp.float32),
    )(acc, s)


def kernel(h, edge_index, edge_attr, W, Wf, bf, a):
    src = edge_index[0]
    dst = edge_index[1]
    npad_e = EPAD - E
    pad_ids = jnp.arange(npad_e, dtype=jnp.int32)
    src_p = jnp.concatenate([src, (pad_ids * 131) % N])
    dst_p = jnp.concatenate([dst, N + (pad_ids % (N2 - N))])
    ea_t = jnp.transpose(edge_attr)
    ea_t = jnp.pad(ea_t, ((0, 0), (0, npad_e)))
    h_p = jnp.pad(h, ((0, N2 - N), (0, 0)))

    z, es, ed = _prep_node(h_p, W, a)
    ef = _prep_edge(ea_t, Wf, bf, a)
    z_flat = z.reshape(NUM_HEADS * N2, OUT_DIM)
    ef_flat = ef.reshape(NUM_HEADS * EPAD)

    zacc = jnp.zeros((NUM_HEADS * N2, OUT_DIM), jnp.float32)
    zs = jnp.zeros((NUM_HEADS * N2,), jnp.float32)
    acc, s = _sc_edge(src_p, dst_p, ef_flat, es, ed, z_flat, zacc, zs)

    out = _finish(acc.reshape(2, NUM_HEADS, N2, OUT_DIM),
                  s.reshape(2, NUM_HEADS, N2))
    return out[:N]


# speedup vs baseline: 14.2272x; 1.1370x over previous
import jax
import jax.numpy as jnp
from jax import lax
from jax.experimental import pallas as pl
from jax.experimental.pallas import tpu as pltpu
from jax.experimental.pallas import tpu_sc as plsc

N = 10000
E = 320000
IN_DIM = 128
OUT_DIM = 32
NUM_HEADS = 4
FEAT = 4

N2 = 10240
NW = 32
ESUB = 10240
EPAD = NW * ESUB
C = 512
K = C // 128
NB = 1024
EB = EPAD // 10


def _prep_node_body(h_ref, w_ref, a_ref, z_ref, es_ref, ed_ref):
    hb = h_ref[...]
    for i in range(NUM_HEADS):
        z = jnp.dot(hb, w_ref[i], preferred_element_type=jnp.float32)
        z_ref[i] = z
        a1 = a_ref[i, 0:OUT_DIM]
        a2 = a_ref[i, OUT_DIM:2 * OUT_DIM]
        es_ref[i] = jnp.sum(z * a1[None, :], axis=1)
        ed_ref[i] = jnp.sum(z * a2[None, :], axis=1)


def _prep_node(h_p, W, a):
    return pl.pallas_call(
        _prep_node_body,
        grid=(N2 // NB,),
        in_specs=[
            pl.BlockSpec((NB, IN_DIM), lambda r: (r, 0)),
            pl.BlockSpec((NUM_HEADS, IN_DIM, OUT_DIM), lambda r: (0, 0, 0)),
            pl.BlockSpec((NUM_HEADS, 3 * OUT_DIM), lambda r: (0, 0)),
        ],
        out_specs=(
            pl.BlockSpec((NUM_HEADS, NB, OUT_DIM), lambda r: (0, r, 0)),
            pl.BlockSpec((NUM_HEADS, NB), lambda r: (0, r)),
            pl.BlockSpec((NUM_HEADS, NB), lambda r: (0, r)),
        ),
        out_shape=(
            jax.ShapeDtypeStruct((NUM_HEADS, N2, OUT_DIM), jnp.float32),
            jax.ShapeDtypeStruct((NUM_HEADS, N2), jnp.float32),
            jax.ShapeDtypeStruct((NUM_HEADS, N2), jnp.float32),
        ),
    )(h_p, W, a)


def _prep_edge_body(eat_ref, wf_ref, bf_ref, a_ref, ef_ref):
    for i in range(NUM_HEADS):
        a3 = a_ref[i, 2 * OUT_DIM:3 * OUT_DIM]
        ef = jnp.full((EB,), jnp.sum(bf_ref[i] * a3), jnp.float32)
        for k in range(FEAT):
            ef = ef + eat_ref[k] * jnp.sum(wf_ref[i, k] * a3)
        ef_ref[i] = ef


def _prep_edge(ea_t, Wf, bf, a):
    return pl.pallas_call(
        _prep_edge_body,
        grid=(EPAD // EB,),
        in_specs=[
            pl.BlockSpec((FEAT, EB), lambda b: (0, b)),
            pl.BlockSpec((NUM_HEADS, FEAT, OUT_DIM), lambda b: (0, 0, 0)),
            pl.BlockSpec((NUM_HEADS, OUT_DIM), lambda b: (0, 0)),
            pl.BlockSpec((NUM_HEADS, 3 * OUT_DIM), lambda b: (0, 0)),
        ],
        out_specs=pl.BlockSpec((NUM_HEADS, EB), lambda b: (0, b)),
        out_shape=jax.ShapeDtypeStruct((NUM_HEADS, EPAD), jnp.float32),
    )(ea_t, Wf, bf, a)


def _sc_body(sd_hbm, ef_hbm, es_hbm, ed_hbm, z_hbm,
             zacc_hbm, zs_hbm, acc_out, s_out,
             es_v, ed_v, sdb, efb,
             exA, exB, rowsA, rowsB, rsA, rsB, giA, giB, diA, diB,
             acc_sh, s_sh,
             semgA, semgB, semaccA, semaccB, semsA, semsB):
    cid = lax.axis_index("c")
    sid = lax.axis_index("s")
    wid = sid * 2 + cid

    @pl.when(sid == 0)
    def _():
        pltpu.sync_copy(zacc_hbm, acc_sh)
        pltpu.sync_copy(zs_hbm, s_sh)
    plsc.subcore_barrier()

    slotA = (exA, rowsA, rsA, giA, diA, semgA, semaccA, semsA)
    slotB = (exB, rowsB, rsB, giB, diB, semgB, semaccB, semsB)

    def wait_scatter(slot):
        ex, _rows, rs, _gi, di, _sg, sacc, ss = slot
        pltpu.make_async_copy(rs, acc_sh.at[di], sacc).wait()
        pltpu.make_async_copy(ex, s_sh.at[di], ss).wait()

    def fill_and_gather(slot, hh, ob):
        ex_r, rows, _rs, gi, di, sg, _sacc, _ss = slot
        off_h = hh * N2
        for g in range(8):
            o = ob + g * 16
            s16 = sdb[pl.ds(o, 16)]
            d16 = sdb[pl.ds(C + o, 16)]
            e = (plsc.load_gather(es_v, [s16])
                 + plsc.load_gather(ed_v, [d16])
                 + efb[pl.ds(o, 16)])
            e = jnp.where(e < 0.0, e * 0.2, e)
            ex_r[pl.ds(g * 16, 16)] = jnp.exp(e)
            gi[pl.ds(g * 16, 16)] = s16 + off_h
            di[pl.ds(g * 16, 16)] = d16 + off_h
        return pltpu.async_copy(z_hbm.at[gi], rows, sg)

    def scale_and_scatter(slot, gdesc):
        ex_r, rows, rs, _gi, di, _sg, sacc, ss = slot
        gdesc.wait()

        def scale_g(g, carry):
            row16 = lax.iota(jnp.int32, 16) + g * 16
            exg = ex_r[pl.ds(g * 16, 16)]
            for cc in range(OUT_DIM):
                c16 = jnp.full((16,), cc, jnp.int32)
                v = plsc.load_gather(rows, [row16, c16])
                plsc.store_scatter(rs, [row16, c16], v * exg)
            return carry

        lax.fori_loop(0, 8, scale_g, 0)
        pltpu.async_copy(rs, acc_sh.at[di], sacc, add=True)
        pltpu.async_copy(ex_r, s_sh.at[di], ss, add=True)

    def head_body(hh, carry):
        pltpu.sync_copy(es_hbm.at[hh], es_v)
        pltpu.sync_copy(ed_hbm.at[hh], ed_v)

        def chunk_body(ch, carry2):
            base = wid * ESUB + ch * C
            pltpu.sync_copy(sd_hbm.at[pl.ds(2 * base, 2 * C)], sdb)
            pltpu.sync_copy(ef_hbm.at[pl.ds(hh * EPAD + base, C)], efb)

            def pair_body(jj, carry3):
                notfirst = jnp.logical_or(
                    jnp.logical_or(hh > 0, ch > 0), jj > 0)

                @pl.when(notfirst)
                def _():
                    wait_scatter(slotA)
                gA = fill_and_gather(slotA, hh, (2 * jj) * 128)

                @pl.when(notfirst)
                def _():
                    wait_scatter(slotB)
                gB = fill_and_gather(slotB, hh, (2 * jj + 1) * 128)

                scale_and_scatter(slotA, gA)
                scale_and_scatter(slotB, gB)
                return carry3

            return lax.fori_loop(0, K // 2, pair_body, carry2)

        return lax.fori_loop(0, ESUB // C, chunk_body, carry)

    lax.fori_loop(0, NUM_HEADS, head_body, 0)
    wait_scatter(slotA)
    wait_scatter(slotB)
    plsc.subcore_barrier()

    @pl.when(sid == 0)
    def _():
        pltpu.sync_copy(acc_sh, acc_out.at[cid])
        pltpu.sync_copy(s_sh, s_out.at[cid])


_sc_edge = pl.kernel(
    _sc_body,
    out_type=(
        jax.ShapeDtypeStruct((2, NUM_HEADS * N2, OUT_DIM), jnp.float32),
        jax.ShapeDtypeStruct((2, NUM_HEADS * N2), jnp.float32),
    ),
    mesh=plsc.VectorSubcoreMesh(core_axis_name="c", subcore_axis_name="s"),
    compiler_params=pltpu.CompilerParams(needs_layout_passes=False,
                                         use_tc_tiling_on_sc=False),
    scratch_types=[
        pltpu.VMEM((N2,), jnp.float32),
        pltpu.VMEM((N2,), jnp.float32),
        pltpu.VMEM((2 * C,), jnp.int32),
        pltpu.VMEM((C,), jnp.float32),
        pltpu.VMEM((128,), jnp.float32),
        pltpu.VMEM((128,), jnp.float32),
        pltpu.VMEM((128, OUT_DIM), jnp.float32),
        pltpu.VMEM((128, OUT_DIM), jnp.float32),
        pltpu.VMEM((128, OUT_DIM), jnp.float32),
        pltpu.VMEM((128, OUT_DIM), jnp.float32),
        pltpu.VMEM((128,), jnp.int32),
        pltpu.VMEM((128,), jnp.int32),
        pltpu.VMEM((128,), jnp.int32),
        pltpu.VMEM((128,), jnp.int32),
        pltpu.VMEM_SHARED((NUM_HEADS * N2, OUT_DIM), jnp.float32),
        pltpu.VMEM_SHARED((NUM_HEADS * N2,), jnp.float32),
        pltpu.SemaphoreType.DMA,
        pltpu.SemaphoreType.DMA,
        pltpu.SemaphoreType.DMA,
        pltpu.SemaphoreType.DMA,
        pltpu.SemaphoreType.DMA,
        pltpu.SemaphoreType.DMA,
    ],
)


def _finish_body(acc_ref, s_ref, out_ref):
    for i in range(NUM_HEADS):
        num = acc_ref[0, i] + acc_ref[1, i]
        den = s_ref[0, i] + s_ref[1, i]
        out_ref[:, i * OUT_DIM:(i + 1) * OUT_DIM] = (
            num / (den + 1e-9)[:, None])


def _finish(acc, s):
    return pl.pallas_call(
        _finish_body,
        grid=(N2 // NB,),
        in_specs=[
            pl.BlockSpec((2, NUM_HEADS, NB, OUT_DIM), lambda r: (0, 0, r, 0)),
            pl.BlockSpec((2, NUM_HEADS, NB), lambda r: (0, 0, r)),
        ],
        out_specs=pl.BlockSpec((NB, NUM_HEADS * OUT_DIM), lambda r: (r, 0)),
        out_shape=jax.ShapeDtypeStruct((N2, NUM_HEADS * OUT_DIM),
                                       jnp.float32),
    )(acc, s)


def kernel(h, edge_index, edge_attr, W, Wf, bf, a):
    src = edge_index[0]
    dst = edge_index[1]
    npad_e = EPAD - E
    pad_ids = jnp.arange(npad_e, dtype=jnp.int32)
    src_p = jnp.concatenate([src, (pad_ids * 131) % N])
    dst_p = jnp.concatenate([dst, N + (pad_ids % (N2 - N))])
    sd = jnp.stack([src_p.reshape(-1, C), dst_p.reshape(-1, C)],
                   axis=1).reshape(-1)
    ea_t = jnp.transpose(edge_attr)
    ea_t = jnp.pad(ea_t, ((0, 0), (0, npad_e)))
    h_p = jnp.pad(h, ((0, N2 - N), (0, 0)))

    z, es, ed = _prep_node(h_p, W, a)
    ef = _prep_edge(ea_t, Wf, bf, a)
    z_flat = z.reshape(NUM_HEADS * N2, OUT_DIM)
    ef_flat = ef.reshape(NUM_HEADS * EPAD)

    zacc = jnp.zeros((NUM_HEADS * N2, OUT_DIM), jnp.float32)
    zs = jnp.zeros((NUM_HEADS * N2,), jnp.float32)
    acc, s = _sc_edge(sd, ef_flat, es, ed, z_flat, zacc, zs)

    out = _finish(acc.reshape(2, NUM_HEADS, N2, OUT_DIM),
                  s.reshape(2, NUM_HEADS, N2))
    return out[:N]
